# baseline (speedup 1.0000x reference)
import jax
import jax.numpy as jnp
from jax import lax
from jax.experimental import pallas as pl
from jax.experimental.pallas import tpu as pltpu
from jax.experimental.pallas import tpu_sc as plsc

N = 10000
NPAD = 10240
E = 320000
T = 12
TP1 = 10
TP2 = 8
CIN = 128
HID = 128
GCN_OUT = 32
BN = 1000

NTILES = 16
EBLK = 128
EBLK2 = 48
NBLK2 = 424
EPT = NBLK2 * EBLK2
BLKS_PER_TILE = EPT // EBLK
EPAD = NTILES * EPT
HALF = 160
DROWS = NPAD // 16


def _ka_body(xt_ref, w1_ref, b1_ref, gw_ref, dv_ref, lo_ref, hi_ref):
    b1 = b1_ref[...]
    gw = gw_ref[...]
    dv = dv_ref[...]
    for t in range(TP1):
        y = b1
        for k in range(3):
            y = y + lax.dot_general(
                xt_ref[:, t + k, :], w1_ref[k],
                (((1,), (0,)), ((), ())),
                preferred_element_type=jnp.float32)
        h = y[:, :HID] * jax.nn.sigmoid(y[:, HID:])
        xl = lax.dot_general(h, gw, (((1,), (0,)), ((), ())),
                             preferred_element_type=jnp.float32) * dv
        tgt = lo_ref if t < 5 else hi_ref
        c = (t % 5) * GCN_OUT
        tgt[:, c:c + GCN_OUT] = xl


def _kernel_a(xt, w1, b1, gw, dinv2):
    return pl.pallas_call(
        _ka_body,
        grid=(N // BN,),
        in_specs=[
            pl.BlockSpec((BN, T, CIN), lambda i: (i, 0, 0)),
            pl.BlockSpec((3, CIN, 2 * HID), lambda i: (0, 0, 0)),
            pl.BlockSpec((1, 2 * HID), lambda i: (0, 0)),
            pl.BlockSpec((CIN, GCN_OUT), lambda i: (0, 0)),
            pl.BlockSpec((BN, 1), lambda i: (i, 0)),
        ],
        out_specs=[
            pl.BlockSpec((BN, HALF), lambda i: (i, 0)),
            pl.BlockSpec((BN, HALF), lambda i: (i, 0)),
        ],
        out_shape=[
            jax.ShapeDtypeStruct((N, HALF), jnp.float32),
            jax.ShapeDtypeStruct((N, HALF), jnp.float32),
        ],
    )(xt, w1, b1, gw, dinv2)


def _kb1_body(dstq_hbm, ew16_hbm, deg0_hbm, deg1_hbm,
              dstq_ref, ewrows_ref, deg_sh):
    cid = lax.axis_index("c")
    tid = lax.axis_index("s")
    zeros16f = jnp.zeros((16,), jnp.float32)

    pltpu.sync_copy(dstq_hbm.at[tid], dstq_ref)

    def zewrows(r, carry):
        ewrows_ref[r, :] = zeros16f
        return carry
    lax.fori_loop(0, 40, zewrows, 0)
    pltpu.sync_copy(ewrows_ref.at[pl.ds(0, 40)],
                    deg_sh.at[pl.ds(tid * 40, 40)])
    plsc.subcore_barrier()

    def deg_blk(j, carry):
        pltpu.sync_copy(ew16_hbm.at[tid, j], ewrows_ref)
        pltpu.sync_copy(ewrows_ref, deg_sh.at[dstq_ref.at[j]], add=True)
        return carry
    lax.fori_loop(cid * 79, 79 + cid * 80, deg_blk, 0)
    plsc.subcore_barrier()

    sl = pl.ds(tid * 40, 40)
    pltpu.sync_copy(deg_sh.at[sl], ewrows_ref.at[pl.ds(0, 40)])

    @pl.when(cid == 0)
    def _():
        pltpu.sync_copy(ewrows_ref.at[pl.ds(0, 40)], deg0_hbm.at[sl])

    @pl.when(cid == 1)
    def _():
        pltpu.sync_copy(ewrows_ref.at[pl.ds(0, 40)], deg1_hbm.at[sl])


def _kernel_b1(dstq, ew16):
    mesh = plsc.VectorSubcoreMesh(core_axis_name="c", subcore_axis_name="s")
    return pl.kernel(
        _kb1_body,
        out_type=[
            jax.ShapeDtypeStruct((DROWS, 16), jnp.float32),
            jax.ShapeDtypeStruct((DROWS, 16), jnp.float32),
        ],
        mesh=mesh,
        compiler_params=pltpu.CompilerParams(needs_layout_passes=False,
                                             use_tc_tiling_on_sc=False),
        scratch_types=[
            pltpu.VMEM((BLKS_PER_TILE, EBLK), jnp.int32),
            pltpu.VMEM((EBLK, 16), jnp.float32),
            pltpu.VMEM_SHARED((DROWS, 16), jnp.float32),
        ],
    )(dstq, ew16)


def _kd_body(d0_ref, d1_ref, dinv_ref):
    dinv_ref[...] = lax.rsqrt(d0_ref[...] + d1_ref[...] + 1.0)


def _kernel_d(deg0, deg1):
    return pl.pallas_call(
        _kd_body,
        out_shape=jax.ShapeDtypeStruct((DROWS, 16), jnp.float32),
    )(deg0, deg1)


def _kb2_body(edata_hbm, xl_hbm, acc_hbm,
              edata_ref, rows_ref, acc_sh, sem_idx, sem_row, sem_sc):
    cid = lax.axis_index("c")
    tid = lax.axis_index("s")
    zeros16f = jnp.zeros((16,), jnp.float32)
    goff = cid * N
    NG = EBLK2 // 16

    def zrows(r, carry):
        for jj in range(10):
            rows_ref[0, r, pl.ds(jj * 16, 16)] = zeros16f
        return carry
    lax.fori_loop(0, EBLK2, zrows, 0)
    base = tid * 640
    for i in range(13):
        pltpu.sync_copy(rows_ref.at[0], acc_sh.at[pl.ds(base + i * 48, 48)])
    pltpu.sync_copy(rows_ref.at[0].at[pl.ds(0, 16)],
                    acc_sh.at[pl.ds(base + 624, 16)])
    plsc.subcore_barrier()

    def fetch(jb):
        r = lax.rem(jb, 6)
        pltpu.async_copy(edata_hbm.at[tid, jb], edata_ref.at[r],
                         sem_idx.at[r])

    def fetch_wait(jb):
        r = lax.rem(jb, 6)
        pltpu.make_async_copy(edata_hbm.at[tid, jb], edata_ref.at[r],
                              sem_idx.at[r]).wait()

    def gather_issue(jb):
        r = lax.rem(jb, 6)
        b = lax.rem(jb, 3)
        for g in range(NG):
            sl = pl.ds(g * 16, 16)
            edata_ref[r, 0, sl] = edata_ref[r, 0, sl] + goff
        pltpu.async_copy(xl_hbm.at[edata_ref.at[r, 0]], rows_ref.at[b],
                         sem_row.at[b])

    def gather_wait(jb):
        r = lax.rem(jb, 6)
        b = lax.rem(jb, 3)
        pltpu.make_async_copy(xl_hbm.at[edata_ref.at[r, 0]], rows_ref.at[b],
                              sem_row.at[b]).wait()

    def scatter_issue(jb):
        r = lax.rem(jb, 6)
        b = lax.rem(jb, 3)
        pltpu.async_copy(rows_ref.at[b], acc_sh.at[edata_ref.at[r, 1]],
                         sem_sc.at[b], add=True)

    def scatter_wait(jb):
        r = lax.rem(jb, 6)
        b = lax.rem(jb, 3)
        pltpu.make_async_copy(rows_ref.at[b], acc_sh.at[edata_ref.at[r, 1]],
                              sem_sc.at[b]).wait()

    fetch(0)
    fetch(1)
    fetch(2)
    fetch_wait(0)
    gather_issue(0)
    fetch_wait(1)
    gather_issue(1)

    def edge_blk(j, carry):
        r = lax.rem(j, 6)
        b = lax.rem(j, 3)
        gather_wait(j)

        def scale(g, c2):
            ewv = plsc.bitcast(edata_ref[r, 2, pl.ds(g * 16, 16)],
                               jnp.float32)
            for lane in range(16):
                s = ewv[lane]
                row = g * 16 + lane
                for jj in range(10):
                    sl2 = pl.ds(jj * 16, 16)
                    rows_ref[b, row, sl2] = rows_ref[b, row, sl2] * s
            return c2
        lax.fori_loop(0, NG, scale, 0)
        scatter_issue(j)

        @pl.when(j + 2 < NBLK2)
        def _():
            fetch_wait(j + 2)

            @pl.when(j >= 1)
            def _():
                scatter_wait(j - 1)
            gather_issue(j + 2)

            @pl.when(j + 3 < NBLK2)
            def _():
                fetch(j + 3)
        return carry
    lax.fori_loop(0, NBLK2, edge_blk, 0)

    for jb in range(NBLK2 - 3, NBLK2):
        scatter_wait(jb)
    plsc.subcore_barrier()

    hbase = cid * NPAD + tid * 640
    for i in range(13):
        pltpu.sync_copy(acc_sh.at[pl.ds(base + i * 48, 48)], rows_ref.at[0])
        pltpu.sync_copy(rows_ref.at[0],
                        acc_hbm.at[pl.ds(hbase + i * 48, 48)])
    pltpu.sync_copy(acc_sh.at[pl.ds(base + 624, 16)],
                    rows_ref.at[0].at[pl.ds(0, 16)])
    pltpu.sync_copy(rows_ref.at[0].at[pl.ds(0, 16)],
                    acc_hbm.at[pl.ds(hbase + 624, 16)])


def _kernel_b2(edata, xl_cat):
    mesh = plsc.VectorSubcoreMesh(core_axis_name="c", subcore_axis_name="s")
    return pl.kernel(
        _kb2_body,
        out_type=[
            jax.ShapeDtypeStruct((2 * NPAD, HALF), jnp.float32),
        ],
        mesh=mesh,
        compiler_params=pltpu.CompilerParams(needs_layout_passes=False,
                                             use_tc_tiling_on_sc=False),
        scratch_types=[
            pltpu.VMEM((6, 3, EBLK2), jnp.int32),
            pltpu.VMEM((3, EBLK2, HALF), jnp.float32),
            pltpu.VMEM_SHARED((NPAD, HALF), jnp.float32),
            pltpu.SemaphoreType.DMA((6,)),
            pltpu.SemaphoreType.DMA((3,)),
            pltpu.SemaphoreType.DMA((3,)),
        ],
    )(edata, xl_cat)


def _kc_body(acc_ref, xl_ref, dinv_ref, gb_ref, w2_ref, b2_ref, out_ref):
    dv = dinv_ref[...]
    g = jnp.maximum(dv * (acc_ref[...] + xl_ref[...]) + gb_ref[...], 0.0)
    z = lax.dot_general(g, w2_ref[...], (((1,), (0,)), ((), ())),
                        preferred_element_type=jnp.float32)
    b2 = b2_ref[...]
    for t in range(TP2):
        zt = z[:, t * 256:(t + 1) * 256] + b2
        out_ref[t] = zt[:, :HID] * jax.nn.sigmoid(zt[:, HID:])


def _kernel_c(acc320, xl320, dinv2, gb320, w2big, b2):
    return pl.pallas_call(
        _kc_body,
        grid=(N // BN,),
        in_specs=[
            pl.BlockSpec((BN, 2 * HALF), lambda i: (i, 0)),
            pl.BlockSpec((BN, 2 * HALF), lambda i: (i, 0)),
            pl.BlockSpec((BN, 1), lambda i: (i, 0)),
            pl.BlockSpec((1, 2 * HALF), lambda i: (0, 0)),
            pl.BlockSpec((2 * HALF, TP2 * 256), lambda i: (0, 0)),
            pl.BlockSpec((1, 256), lambda i: (0, 0)),
        ],
        out_specs=pl.BlockSpec((TP2, BN, HID), lambda i: (0, i, 0)),
        out_shape=jax.ShapeDtypeStruct((TP2, N, HID), jnp.float32),
    )(acc320, xl320, dinv2, gb320, w2big, b2)


@jax.jit
def kernel(x, edge_index, edge_attr, batch, conv1_w, conv1_b, gcn_w, gcn_b,
           conv2_w, conv2_b):
    del batch
    xt = jnp.transpose(x, (0, 2, 1))
    w1 = jnp.transpose(conv1_w, (2, 1, 0))
    b1 = conv1_b[None, :]

    pad = EPAD - E
    fill = jnp.arange(pad, dtype=jnp.int32) % N
    src_f = jnp.concatenate([edge_index[0], fill])
    dst_f = jnp.concatenate([edge_index[1], fill])
    ew_f = jnp.concatenate([edge_attr, jnp.zeros((pad,), jnp.float32)])
    srcp = src_f.reshape(NTILES, NBLK2, EBLK2)
    dstp = dst_f.reshape(NTILES, NBLK2, EBLK2)
    ewp = lax.bitcast_convert_type(ew_f, jnp.int32).reshape(
        NTILES, NBLK2, EBLK2)
    edata = jnp.stack([srcp, dstp, ewp], axis=2)
    dstq = (dst_f >> 4).reshape(NTILES, BLKS_PER_TILE, EBLK)
    ew16 = (ew_f[:, None] * jax.nn.one_hot(dst_f & 15, 16, dtype=jnp.float32)
            ).reshape(NTILES, BLKS_PER_TILE, EBLK, 16)

    deg0, deg1 = _kernel_b1(dstq, ew16)
    dinv = _kernel_d(deg0, deg1).reshape(NPAD)
    dinv2 = dinv[:N, None]

    xl_lo, xl_hi = _kernel_a(xt, w1, b1, gcn_w, dinv2)
    xl_cat = jnp.concatenate([xl_lo, xl_hi], axis=0)
    (acc_cat,) = _kernel_b2(edata, xl_cat)

    acc320 = jnp.concatenate([acc_cat[:N], acc_cat[NPAD:NPAD + N]], axis=1)
    xl320 = jnp.concatenate([xl_lo, xl_hi], axis=1)
    gb320 = jnp.tile(gcn_b, TP1)[None, :]

    w2t = jnp.transpose(conv2_w, (2, 1, 0))
    w2big = jnp.zeros((2 * HALF, TP2 * 256), jnp.float32)
    for t in range(TP2):
        for k in range(3):
            w2big = w2big.at[(t + k) * GCN_OUT:(t + k + 1) * GCN_OUT,
                             t * 256:(t + 1) * 256].set(w2t[k])

    out8 = _kernel_c(acc320, xl320, dinv[:N, None], gb320, w2big,
                     conv2_b[None, :])
    return jnp.transpose(out8, (1, 2, 0))


# speedup vs baseline: 1.6832x; 1.6832x over previous
import jax
import jax.numpy as jnp
from jax import lax
from jax.experimental import pallas as pl
from jax.experimental.pallas import tpu as pltpu
from jax.experimental.pallas import tpu_sc as plsc

N = 10000
NPAD = 10240
E = 320000
T = 12
TP1 = 10
TP2 = 8
CIN = 128
HID = 128
GCN_OUT = 32
BN = 1000

NTILES = 16
EBLK = 128
EBLK2 = 48
NBLK2 = 424
EPT = NBLK2 * EBLK2
BLKS_PER_TILE = EPT // EBLK
EPAD = NTILES * EPT
HALF = 160
DROWS = NPAD // 16


def _ka_body(xt_ref, w1_ref, b1_ref, gw_ref, dv_ref, lo_ref, hi_ref):
    b1 = b1_ref[...]
    gw = gw_ref[...]
    dv = dv_ref[...]
    for t in range(TP1):
        y = b1
        for k in range(3):
            y = y + lax.dot_general(
                xt_ref[:, t + k, :], w1_ref[k],
                (((1,), (0,)), ((), ())),
                preferred_element_type=jnp.float32)
        h = y[:, :HID] * jax.nn.sigmoid(y[:, HID:])
        xl = lax.dot_general(h, gw, (((1,), (0,)), ((), ())),
                             preferred_element_type=jnp.float32) * dv
        tgt = lo_ref if t < 5 else hi_ref
        c = (t % 5) * GCN_OUT
        tgt[:, c:c + GCN_OUT] = xl


def _kernel_a(xt, w1, b1, gw, dinv2):
    return pl.pallas_call(
        _ka_body,
        grid=(N // BN,),
        in_specs=[
            pl.BlockSpec((BN, T, CIN), lambda i: (i, 0, 0)),
            pl.BlockSpec((3, CIN, 2 * HID), lambda i: (0, 0, 0)),
            pl.BlockSpec((1, 2 * HID), lambda i: (0, 0)),
            pl.BlockSpec((CIN, GCN_OUT), lambda i: (0, 0)),
            pl.BlockSpec((BN, 1), lambda i: (i, 0)),
        ],
        out_specs=[
            pl.BlockSpec((BN, HALF), lambda i: (i, 0)),
            pl.BlockSpec((BN, HALF), lambda i: (i, 0)),
        ],
        out_shape=[
            jax.ShapeDtypeStruct((N, HALF), jnp.float32),
            jax.ShapeDtypeStruct((N, HALF), jnp.float32),
        ],
    )(xt, w1, b1, gw, dinv2)


def _kb1_body(dstq_hbm, ew16_hbm, deg0_hbm, deg1_hbm,
              dstq_ref, ewrows_ref, deg_sh):
    cid = lax.axis_index("c")
    tid = lax.axis_index("s")
    zeros16f = jnp.zeros((16,), jnp.float32)

    pltpu.sync_copy(dstq_hbm.at[tid], dstq_ref)

    def zewrows(r, carry):
        ewrows_ref[r, :] = zeros16f
        return carry
    lax.fori_loop(0, 40, zewrows, 0)
    pltpu.sync_copy(ewrows_ref.at[pl.ds(0, 40)],
                    deg_sh.at[pl.ds(tid * 40, 40)])
    plsc.subcore_barrier()

    def deg_blk(j, carry):
        pltpu.sync_copy(ew16_hbm.at[tid, j], ewrows_ref)
        pltpu.sync_copy(ewrows_ref, deg_sh.at[dstq_ref.at[j]], add=True)
        return carry
    lax.fori_loop(cid * 79, 79 + cid * 80, deg_blk, 0)
    plsc.subcore_barrier()

    sl = pl.ds(tid * 40, 40)
    pltpu.sync_copy(deg_sh.at[sl], ewrows_ref.at[pl.ds(0, 40)])

    @pl.when(cid == 0)
    def _():
        pltpu.sync_copy(ewrows_ref.at[pl.ds(0, 40)], deg0_hbm.at[sl])

    @pl.when(cid == 1)
    def _():
        pltpu.sync_copy(ewrows_ref.at[pl.ds(0, 40)], deg1_hbm.at[sl])


def _kernel_b1(dstq, ew16):
    mesh = plsc.VectorSubcoreMesh(core_axis_name="c", subcore_axis_name="s")
    return pl.kernel(
        _kb1_body,
        out_type=[
            jax.ShapeDtypeStruct((DROWS, 16), jnp.float32),
            jax.ShapeDtypeStruct((DROWS, 16), jnp.float32),
        ],
        mesh=mesh,
        compiler_params=pltpu.CompilerParams(needs_layout_passes=False,
                                             use_tc_tiling_on_sc=False),
        scratch_types=[
            pltpu.VMEM((BLKS_PER_TILE, EBLK), jnp.int32),
            pltpu.VMEM((EBLK, 16), jnp.float32),
            pltpu.VMEM_SHARED((DROWS, 16), jnp.float32),
        ],
    )(dstq, ew16)


def _kd_body(d0_ref, d1_ref, dinv_ref):
    dinv_ref[...] = lax.rsqrt(d0_ref[...] + d1_ref[...] + 1.0)


def _kernel_d(deg0, deg1):
    return pl.pallas_call(
        _kd_body,
        out_shape=jax.ShapeDtypeStruct((DROWS, 16), jnp.float32),
    )(deg0, deg1)


def _kb2_body(edata_hbm, xl_hbm, acc_hbm,
              edata_ref, rows_ref, acc_sh, sem_idx, sem_row, sem_sc):
    cid = lax.axis_index("c")
    tid = lax.axis_index("s")
    zeros16f = jnp.zeros((16,), jnp.float32)
    goff = cid * N
    NG = EBLK2 // 16

    def zrows(r, carry):
        for jj in range(10):
            rows_ref[0, r, pl.ds(jj * 16, 16)] = zeros16f
        return carry
    lax.fori_loop(0, EBLK2, zrows, 0)
    base = tid * 640
    for i in range(13):
        pltpu.sync_copy(rows_ref.at[0], acc_sh.at[pl.ds(base + i * 48, 48)])
    pltpu.sync_copy(rows_ref.at[0].at[pl.ds(0, 16)],
                    acc_sh.at[pl.ds(base + 624, 16)])
    plsc.subcore_barrier()

    def fetch(jb):
        r = lax.rem(jb, 6)
        pltpu.async_copy(edata_hbm.at[tid, jb], edata_ref.at[r],
                         sem_idx.at[r])

    def fetch_wait(jb):
        r = lax.rem(jb, 6)
        pltpu.make_async_copy(edata_hbm.at[tid, jb], edata_ref.at[r],
                              sem_idx.at[r]).wait()

    def gather_issue(jb):
        r = lax.rem(jb, 6)
        b = lax.rem(jb, 3)
        for g in range(NG):
            sl = pl.ds(g * 16, 16)
            edata_ref[r, 0, sl] = edata_ref[r, 0, sl] + goff
        pltpu.async_copy(xl_hbm.at[edata_ref.at[r, 0]], rows_ref.at[b],
                         sem_row.at[b])

    def gather_wait(jb):
        r = lax.rem(jb, 6)
        b = lax.rem(jb, 3)
        pltpu.make_async_copy(xl_hbm.at[edata_ref.at[r, 0]], rows_ref.at[b],
                              sem_row.at[b]).wait()

    def scatter_issue(jb):
        r = lax.rem(jb, 6)
        b = lax.rem(jb, 3)
        pltpu.async_copy(rows_ref.at[b], acc_sh.at[edata_ref.at[r, 1]],
                         sem_sc.at[b], add=True)

    def scatter_wait(jb):
        r = lax.rem(jb, 6)
        b = lax.rem(jb, 3)
        pltpu.make_async_copy(rows_ref.at[b], acc_sh.at[edata_ref.at[r, 1]],
                              sem_sc.at[b]).wait()

    fetch(0)
    fetch(1)
    fetch(2)
    fetch_wait(0)
    gather_issue(0)
    fetch_wait(1)
    gather_issue(1)

    def edge_blk(j, carry):
        r = lax.rem(j, 6)
        b = lax.rem(j, 3)
        gather_wait(j)

        for g in range(NG):
            ewv = plsc.bitcast(edata_ref[r, 2, pl.ds(g * 16, 16)],
                               jnp.float32)
            for lane in range(16):
                s = ewv[lane]
                row = g * 16 + lane
                for jj in range(10):
                    sl2 = pl.ds(jj * 16, 16)
                    rows_ref[b, row, sl2] = rows_ref[b, row, sl2] * s
        scatter_issue(j)

        @pl.when(j + 2 < NBLK2)
        def _():
            fetch_wait(j + 2)

            @pl.when(j >= 1)
            def _():
                scatter_wait(j - 1)
            gather_issue(j + 2)

            @pl.when(j + 3 < NBLK2)
            def _():
                fetch(j + 3)
        return carry
    lax.fori_loop(0, NBLK2, edge_blk, 0)

    for jb in range(NBLK2 - 3, NBLK2):
        scatter_wait(jb)
    plsc.subcore_barrier()

    hbase = cid * NPAD + tid * 640
    for i in range(13):
        pltpu.sync_copy(acc_sh.at[pl.ds(base + i * 48, 48)], rows_ref.at[0])
        pltpu.sync_copy(rows_ref.at[0],
                        acc_hbm.at[pl.ds(hbase + i * 48, 48)])
    pltpu.sync_copy(acc_sh.at[pl.ds(base + 624, 16)],
                    rows_ref.at[0].at[pl.ds(0, 16)])
    pltpu.sync_copy(rows_ref.at[0].at[pl.ds(0, 16)],
                    acc_hbm.at[pl.ds(hbase + 624, 16)])


def _kernel_b2(edata, xl_cat):
    mesh = plsc.VectorSubcoreMesh(core_axis_name="c", subcore_axis_name="s")
    return pl.kernel(
        _kb2_body,
        out_type=[
            jax.ShapeDtypeStruct((2 * NPAD, HALF), jnp.float32),
        ],
        mesh=mesh,
        compiler_params=pltpu.CompilerParams(needs_layout_passes=False,
                                             use_tc_tiling_on_sc=False),
        scratch_types=[
            pltpu.VMEM((6, 3, EBLK2), jnp.int32),
            pltpu.VMEM((3, EBLK2, HALF), jnp.float32),
            pltpu.VMEM_SHARED((NPAD, HALF), jnp.float32),
            pltpu.SemaphoreType.DMA((6,)),
            pltpu.SemaphoreType.DMA((3,)),
            pltpu.SemaphoreType.DMA((3,)),
        ],
    )(edata, xl_cat)


def _kc_body(acc_ref, xl_ref, dinv_ref, gb_ref, w2_ref, b2_ref, out_ref):
    dv = dinv_ref[...]
    g = jnp.maximum(dv * (acc_ref[...] + xl_ref[...]) + gb_ref[...], 0.0)
    z = lax.dot_general(g, w2_ref[...], (((1,), (0,)), ((), ())),
                        preferred_element_type=jnp.float32)
    b2 = b2_ref[...]
    for t in range(TP2):
        zt = z[:, t * 256:(t + 1) * 256] + b2
        out_ref[t] = zt[:, :HID] * jax.nn.sigmoid(zt[:, HID:])


def _kernel_c(acc320, xl320, dinv2, gb320, w2big, b2):
    return pl.pallas_call(
        _kc_body,
        grid=(N // BN,),
        in_specs=[
            pl.BlockSpec((BN, 2 * HALF), lambda i: (i, 0)),
            pl.BlockSpec((BN, 2 * HALF), lambda i: (i, 0)),
            pl.BlockSpec((BN, 1), lambda i: (i, 0)),
            pl.BlockSpec((1, 2 * HALF), lambda i: (0, 0)),
            pl.BlockSpec((2 * HALF, TP2 * 256), lambda i: (0, 0)),
            pl.BlockSpec((1, 256), lambda i: (0, 0)),
        ],
        out_specs=pl.BlockSpec((TP2, BN, HID), lambda i: (0, i, 0)),
        out_shape=jax.ShapeDtypeStruct((TP2, N, HID), jnp.float32),
    )(acc320, xl320, dinv2, gb320, w2big, b2)


@jax.jit
def kernel(x, edge_index, edge_attr, batch, conv1_w, conv1_b, gcn_w, gcn_b,
           conv2_w, conv2_b):
    del batch
    xt = jnp.transpose(x, (0, 2, 1))
    w1 = jnp.transpose(conv1_w, (2, 1, 0))
    b1 = conv1_b[None, :]

    pad = EPAD - E
    fill = jnp.arange(pad, dtype=jnp.int32) % N
    src_f = jnp.concatenate([edge_index[0], fill])
    dst_f = jnp.concatenate([edge_index[1], fill])
    ew_f = jnp.concatenate([edge_attr, jnp.zeros((pad,), jnp.float32)])
    srcp = src_f.reshape(NTILES, NBLK2, EBLK2)
    dstp = dst_f.reshape(NTILES, NBLK2, EBLK2)
    ewp = lax.bitcast_convert_type(ew_f, jnp.int32).reshape(
        NTILES, NBLK2, EBLK2)
    edata = jnp.stack([srcp, dstp, ewp], axis=2)
    dstq = (dst_f >> 4).reshape(NTILES, BLKS_PER_TILE, EBLK)
    ew16 = (ew_f[:, None] * jax.nn.one_hot(dst_f & 15, 16, dtype=jnp.float32)
            ).reshape(NTILES, BLKS_PER_TILE, EBLK, 16)

    deg0, deg1 = _kernel_b1(dstq, ew16)
    dinv = _kernel_d(deg0, deg1).reshape(NPAD)
    dinv2 = dinv[:N, None]

    xl_lo, xl_hi = _kernel_a(xt, w1, b1, gcn_w, dinv2)
    xl_cat = jnp.concatenate([xl_lo, xl_hi], axis=0)
    (acc_cat,) = _kernel_b2(edata, xl_cat)

    acc320 = jnp.concatenate([acc_cat[:N], acc_cat[NPAD:NPAD + N]], axis=1)
    xl320 = jnp.concatenate([xl_lo, xl_hi], axis=1)
    gb320 = jnp.tile(gcn_b, TP1)[None, :]

    w2t = jnp.transpose(conv2_w, (2, 1, 0))
    w2big = jnp.zeros((2 * HALF, TP2 * 256), jnp.float32)
    for t in range(TP2):
        for k in range(3):
            w2big = w2big.at[(t + k) * GCN_OUT:(t + k + 1) * GCN_OUT,
                             t * 256:(t + 1) * 256].set(w2t[k])

    out8 = _kernel_c(acc320, xl320, dinv[:N, None], gb320, w2big,
                     conv2_b[None, :])
    return jnp.transpose(out8, (1, 2, 0))


# speedup vs baseline: 1.7804x; 1.0577x over previous
import jax
import jax.numpy as jnp
from jax import lax
from jax.experimental import pallas as pl
from jax.experimental.pallas import tpu as pltpu
from jax.experimental.pallas import tpu_sc as plsc

N = 10000
NPAD = 10240
ACCR = 10000
E = 320000
T = 12
TP1 = 10
TP2 = 8
CIN = 128
HID = 128
GCN_OUT = 32
BN = 1000

NTILES = 16
EBLK = 128
EBLK2 = 48
NBLK2 = 424
EPT = NBLK2 * EBLK2
BLKS_PER_TILE = EPT // EBLK
EPAD = NTILES * EPT
HALF = 160
DROWS = NPAD // 16


def _ka_body(xt_ref, w1_ref, b1_ref, gw_ref, dv_ref, lo_ref, hi_ref):
    b1 = b1_ref[...]
    gw = gw_ref[...]
    dv = dv_ref[...]
    for t in range(TP1):
        y = b1
        for k in range(3):
            y = y + lax.dot_general(
                xt_ref[:, t + k, :], w1_ref[k],
                (((1,), (0,)), ((), ())),
                preferred_element_type=jnp.float32)
        h = y[:, :HID] * jax.nn.sigmoid(y[:, HID:])
        xl = lax.dot_general(h, gw, (((1,), (0,)), ((), ())),
                             preferred_element_type=jnp.float32) * dv
        tgt = lo_ref if t < 5 else hi_ref
        c = (t % 5) * GCN_OUT
        tgt[:, c:c + GCN_OUT] = xl


def _kernel_a(xt, w1, b1, gw, dinv2):
    return pl.pallas_call(
        _ka_body,
        grid=(N // BN,),
        in_specs=[
            pl.BlockSpec((BN, T, CIN), lambda i: (i, 0, 0)),
            pl.BlockSpec((3, CIN, 2 * HID), lambda i: (0, 0, 0)),
            pl.BlockSpec((1, 2 * HID), lambda i: (0, 0)),
            pl.BlockSpec((CIN, GCN_OUT), lambda i: (0, 0)),
            pl.BlockSpec((BN, 1), lambda i: (i, 0)),
        ],
        out_specs=[
            pl.BlockSpec((BN, HALF), lambda i: (i, 0)),
            pl.BlockSpec((BN, HALF), lambda i: (i, 0)),
        ],
        out_shape=[
            jax.ShapeDtypeStruct((N, HALF), jnp.float32),
            jax.ShapeDtypeStruct((N, HALF), jnp.float32),
        ],
    )(xt, w1, b1, gw, dinv2)


def _kb1_body(dstq_hbm, ew16_hbm, deg0_hbm, deg1_hbm,
              dstq_ref, ewrows_ref, deg_sh):
    cid = lax.axis_index("c")
    tid = lax.axis_index("s")
    zeros16f = jnp.zeros((16,), jnp.float32)

    pltpu.sync_copy(dstq_hbm.at[tid], dstq_ref)

    def zewrows(r, carry):
        ewrows_ref[r, :] = zeros16f
        return carry
    lax.fori_loop(0, 40, zewrows, 0)
    pltpu.sync_copy(ewrows_ref.at[pl.ds(0, 40)],
                    deg_sh.at[pl.ds(tid * 40, 40)])
    plsc.subcore_barrier()

    def deg_blk(j, carry):
        pltpu.sync_copy(ew16_hbm.at[tid, j], ewrows_ref)
        pltpu.sync_copy(ewrows_ref, deg_sh.at[dstq_ref.at[j]], add=True)
        return carry
    lax.fori_loop(cid * 79, 79 + cid * 80, deg_blk, 0)
    plsc.subcore_barrier()

    sl = pl.ds(tid * 40, 40)
    pltpu.sync_copy(deg_sh.at[sl], ewrows_ref.at[pl.ds(0, 40)])

    @pl.when(cid == 0)
    def _():
        pltpu.sync_copy(ewrows_ref.at[pl.ds(0, 40)], deg0_hbm.at[sl])

    @pl.when(cid == 1)
    def _():
        pltpu.sync_copy(ewrows_ref.at[pl.ds(0, 40)], deg1_hbm.at[sl])


def _kernel_b1(dstq, ew16):
    mesh = plsc.VectorSubcoreMesh(core_axis_name="c", subcore_axis_name="s")
    return pl.kernel(
        _kb1_body,
        out_type=[
            jax.ShapeDtypeStruct((DROWS, 16), jnp.float32),
            jax.ShapeDtypeStruct((DROWS, 16), jnp.float32),
        ],
        mesh=mesh,
        compiler_params=pltpu.CompilerParams(needs_layout_passes=False,
                                             use_tc_tiling_on_sc=False),
        scratch_types=[
            pltpu.VMEM((BLKS_PER_TILE, EBLK), jnp.int32),
            pltpu.VMEM((EBLK, 16), jnp.float32),
            pltpu.VMEM_SHARED((DROWS, 16), jnp.float32),
        ],
    )(dstq, ew16)


def _kd_body(d0_ref, d1_ref, dinv_ref):
    dinv_ref[...] = lax.rsqrt(d0_ref[...] + d1_ref[...] + 1.0)


def _kernel_d(deg0, deg1):
    return pl.pallas_call(
        _kd_body,
        out_shape=jax.ShapeDtypeStruct((DROWS, 16), jnp.float32),
    )(deg0, deg1)


def _kb2_body(edata_hbm, xlo_hbm, xhi_hbm, acc_hbm,
              edata_ref, rows_ref, acc_sh, sem_idx, sem_row, sem_sc):
    cid = lax.axis_index("c")
    tid = lax.axis_index("s")
    zeros16f = jnp.zeros((16,), jnp.float32)
    NG = EBLK2 // 16

    def zrows(r, carry):
        for jj in range(10):
            rows_ref[0, r, pl.ds(jj * 16, 16)] = zeros16f
        return carry
    lax.fori_loop(0, EBLK2, zrows, 0)
    base = tid * 625
    for i in range(13):
        pltpu.sync_copy(rows_ref.at[0], acc_sh.at[pl.ds(base + i * 48, 48)])
    pltpu.sync_copy(rows_ref.at[0].at[pl.ds(0, 1)],
                    acc_sh.at[pl.ds(base + 624, 1)])
    plsc.subcore_barrier()

    def fetch(jb):
        r = lax.rem(jb, 6)
        pltpu.async_copy(edata_hbm.at[tid, jb], edata_ref.at[r],
                         sem_idx.at[r])

    def fetch_wait(jb):
        r = lax.rem(jb, 6)
        pltpu.make_async_copy(edata_hbm.at[tid, jb], edata_ref.at[r],
                              sem_idx.at[r]).wait()

    def gather_issue(jb):
        r = lax.rem(jb, 6)
        b = lax.rem(jb, 3)

        @pl.when(cid == 0)
        def _():
            pltpu.async_copy(xlo_hbm.at[edata_ref.at[r, 0]], rows_ref.at[b],
                             sem_row.at[b])

        @pl.when(cid == 1)
        def _():
            pltpu.async_copy(xhi_hbm.at[edata_ref.at[r, 0]], rows_ref.at[b],
                             sem_row.at[b])

    def gather_wait(jb):
        r = lax.rem(jb, 6)
        b = lax.rem(jb, 3)
        pltpu.make_async_copy(xlo_hbm.at[edata_ref.at[r, 0]], rows_ref.at[b],
                              sem_row.at[b]).wait()

    def scatter_issue(jb):
        r = lax.rem(jb, 6)
        b = lax.rem(jb, 3)
        pltpu.async_copy(rows_ref.at[b], acc_sh.at[edata_ref.at[r, 1]],
                         sem_sc.at[b], add=True)

    def scatter_wait(jb):
        r = lax.rem(jb, 6)
        b = lax.rem(jb, 3)
        pltpu.make_async_copy(rows_ref.at[b], acc_sh.at[edata_ref.at[r, 1]],
                              sem_sc.at[b]).wait()

    fetch(0)
    fetch(1)
    fetch(2)
    fetch_wait(0)
    gather_issue(0)
    fetch_wait(1)
    gather_issue(1)

    def edge_blk(j, carry):
        r = lax.rem(j, 6)
        b = lax.rem(j, 3)
        gather_wait(j)

        for g in range(NG):
            ewv = plsc.bitcast(edata_ref[r, 2, pl.ds(g * 16, 16)],
                               jnp.float32)
            for lane in range(16):
                s = ewv[lane]
                row = g * 16 + lane
                for jj in range(10):
                    sl2 = pl.ds(jj * 16, 16)
                    rows_ref[b, row, sl2] = rows_ref[b, row, sl2] * s
        scatter_issue(j)

        @pl.when(j + 2 < NBLK2)
        def _():
            fetch_wait(j + 2)

            @pl.when(j >= 1)
            def _():
                scatter_wait(j - 1)
            gather_issue(j + 2)

            @pl.when(j + 3 < NBLK2)
            def _():
                fetch(j + 3)
        return carry
    lax.fori_loop(0, NBLK2, edge_blk, 0)

    for jb in range(NBLK2 - 3, NBLK2):
        scatter_wait(jb)
    plsc.subcore_barrier()

    hbase = cid * ACCR + tid * 625
    for i in range(13):
        pltpu.sync_copy(acc_sh.at[pl.ds(base + i * 48, 48)], rows_ref.at[0])
        pltpu.sync_copy(rows_ref.at[0],
                        acc_hbm.at[pl.ds(hbase + i * 48, 48)])
    pltpu.sync_copy(acc_sh.at[pl.ds(base + 624, 1)],
                    rows_ref.at[0].at[pl.ds(0, 1)])
    pltpu.sync_copy(rows_ref.at[0].at[pl.ds(0, 1)],
                    acc_hbm.at[pl.ds(hbase + 624, 1)])


def _kernel_b2(edata, xl_lo, xl_hi):
    mesh = plsc.VectorSubcoreMesh(core_axis_name="c", subcore_axis_name="s")
    return pl.kernel(
        _kb2_body,
        out_type=[
            jax.ShapeDtypeStruct((2 * ACCR, HALF), jnp.float32),
        ],
        mesh=mesh,
        compiler_params=pltpu.CompilerParams(needs_layout_passes=False,
                                             use_tc_tiling_on_sc=False),
        scratch_types=[
            pltpu.VMEM((6, 3, EBLK2), jnp.int32),
            pltpu.VMEM((3, EBLK2, HALF), jnp.float32),
            pltpu.VMEM_SHARED((ACCR, HALF), jnp.float32),
            pltpu.SemaphoreType.DMA((6,)),
            pltpu.SemaphoreType.DMA((3,)),
            pltpu.SemaphoreType.DMA((3,)),
        ],
    )(edata, xl_lo, xl_hi)


def _kc_body(alo_ref, ahi_ref, xlo_ref, xhi_ref, dinv_ref, gb_ref, w2_ref,
             b2_ref, out_ref):
    dv = dinv_ref[...]
    gb = gb_ref[...]
    glo = jnp.maximum(dv * (alo_ref[...] + xlo_ref[...]) + gb, 0.0)
    ghi = jnp.maximum(dv * (ahi_ref[...] + xhi_ref[...]) + gb, 0.0)
    z = (lax.dot_general(glo, w2_ref[:HALF], (((1,), (0,)), ((), ())),
                         preferred_element_type=jnp.float32)
         + lax.dot_general(ghi, w2_ref[HALF:], (((1,), (0,)), ((), ())),
                           preferred_element_type=jnp.float32))
    b2 = b2_ref[...]
    for t in range(TP2):
        zt = z[:, t * 256:(t + 1) * 256] + b2
        out_ref[t] = zt[:, :HID] * jax.nn.sigmoid(zt[:, HID:])


def _kernel_c(acc_cat, xl_lo, xl_hi, dinv2, gb160, w2big, b2):
    return pl.pallas_call(
        _kc_body,
        grid=(N // BN,),
        in_specs=[
            pl.BlockSpec((BN, HALF), lambda i: (i, 0)),
            pl.BlockSpec((BN, HALF), lambda i: (i + ACCR // BN, 0)),
            pl.BlockSpec((BN, HALF), lambda i: (i, 0)),
            pl.BlockSpec((BN, HALF), lambda i: (i, 0)),
            pl.BlockSpec((BN, 1), lambda i: (i, 0)),
            pl.BlockSpec((1, HALF), lambda i: (0, 0)),
            pl.BlockSpec((2 * HALF, TP2 * 256), lambda i: (0, 0)),
            pl.BlockSpec((1, 256), lambda i: (0, 0)),
        ],
        out_specs=pl.BlockSpec((TP2, BN, HID), lambda i: (0, i, 0)),
        out_shape=jax.ShapeDtypeStruct((TP2, N, HID), jnp.float32),
    )(acc_cat, acc_cat, xl_lo, xl_hi, dinv2, gb160, w2big, b2)


@jax.jit
def kernel(x, edge_index, edge_attr, batch, conv1_w, conv1_b, gcn_w, gcn_b,
           conv2_w, conv2_b):
    del batch
    xt = jnp.transpose(x, (0, 2, 1))
    w1 = jnp.transpose(conv1_w, (2, 1, 0))
    b1 = conv1_b[None, :]

    pad = EPAD - E
    fill = jnp.arange(pad, dtype=jnp.int32) % N
    src_f = jnp.concatenate([edge_index[0], fill])
    dst_f = jnp.concatenate([edge_index[1], fill])
    ew_f = jnp.concatenate([edge_attr, jnp.zeros((pad,), jnp.float32)])
    srcp = src_f.reshape(NTILES, NBLK2, EBLK2)
    dstp = dst_f.reshape(NTILES, NBLK2, EBLK2)
    ewp = lax.bitcast_convert_type(ew_f, jnp.int32).reshape(
        NTILES, NBLK2, EBLK2)
    edata = jnp.stack([srcp, dstp, ewp], axis=2)
    dstq = (dst_f >> 4).reshape(NTILES, BLKS_PER_TILE, EBLK)
    ew16 = (ew_f[:, None] * jax.nn.one_hot(dst_f & 15, 16, dtype=jnp.float32)
            ).reshape(NTILES, BLKS_PER_TILE, EBLK, 16)

    deg0, deg1 = _kernel_b1(dstq, ew16)
    dinv = _kernel_d(deg0, deg1).reshape(NPAD)
    dinv2 = dinv[:N, None]

    xl_lo, xl_hi = _kernel_a(xt, w1, b1, gcn_w, dinv2)
    (acc_cat,) = _kernel_b2(edata, xl_lo, xl_hi)

    gb160 = jnp.tile(gcn_b, 5)[None, :]

    w2t = jnp.transpose(conv2_w, (2, 1, 0))
    w2big = jnp.zeros((2 * HALF, TP2 * 256), jnp.float32)
    for t in range(TP2):
        for k in range(3):
            w2big = w2big.at[(t + k) * GCN_OUT:(t + k + 1) * GCN_OUT,
                             t * 256:(t + 1) * 256].set(w2t[k])

    out8 = _kernel_c(acc_cat, xl_lo, xl_hi, dinv2, gb160, w2big,
                     conv2_b[None, :])
    return jnp.transpose(out8, (1, 2, 0))


# speedup vs baseline: 2.4891x; 1.3980x over previous
import jax
import jax.numpy as jnp
from jax import lax
from jax.experimental import pallas as pl
from jax.experimental.pallas import tpu as pltpu
from jax.experimental.pallas import tpu_sc as plsc

N = 10000
NPAD = 10240
ACCR = 10000
E = 320000
T = 12
TP1 = 10
TP2 = 8
CIN = 128
HID = 128
GCN_OUT = 32
BN = 1000

NTILES = 16
EBLK = 128
EBLK2 = 48
NBLK2 = 424
EPT = NBLK2 * EBLK2
BLKS_PER_TILE = EPT // EBLK
EPAD = NTILES * EPT
HALF = 160
DROWS = NPAD // 16


def _ka_body(xt_ref, w1_ref, b1_ref, gw_ref, dv_ref, lo_ref, hi_ref):
    b1 = b1_ref[...]
    gw = gw_ref[...]
    dv = dv_ref[...]
    for t in range(TP1):
        y = b1
        for k in range(3):
            y = y + lax.dot_general(
                xt_ref[:, t + k, :], w1_ref[k],
                (((1,), (0,)), ((), ())),
                preferred_element_type=jnp.float32)
        h = y[:, :HID] * jax.nn.sigmoid(y[:, HID:])
        xl = lax.dot_general(h, gw, (((1,), (0,)), ((), ())),
                             preferred_element_type=jnp.float32) * dv
        tgt = lo_ref if t < 5 else hi_ref
        c = (t % 5) * GCN_OUT
        tgt[:, c:c + GCN_OUT] = xl


def _kernel_a(xt, w1, b1, gw, dinv2):
    return pl.pallas_call(
        _ka_body,
        grid=(N // BN,),
        in_specs=[
            pl.BlockSpec((BN, T, CIN), lambda i: (i, 0, 0)),
            pl.BlockSpec((3, CIN, 2 * HID), lambda i: (0, 0, 0)),
            pl.BlockSpec((1, 2 * HID), lambda i: (0, 0)),
            pl.BlockSpec((CIN, GCN_OUT), lambda i: (0, 0)),
            pl.BlockSpec((BN, 1), lambda i: (i, 0)),
        ],
        out_specs=[
            pl.BlockSpec((BN, HALF), lambda i: (i, 0)),
            pl.BlockSpec((BN, HALF), lambda i: (i, 0)),
        ],
        out_shape=[
            jax.ShapeDtypeStruct((N, HALF), jnp.float32),
            jax.ShapeDtypeStruct((N, HALF), jnp.float32),
        ],
    )(xt, w1, b1, gw, dinv2)


def _kb1_body(dst_hbm, ew_hbm, deg0_hbm, deg1_hbm,
              dst_ref, ew_ref, dstq_ref, ewrows_ref, deg_sh):
    cid = lax.axis_index("c")
    tid = lax.axis_index("s")
    zeros16f = jnp.zeros((16,), jnp.float32)
    iota = jnp.arange(16, dtype=jnp.int32)

    pltpu.sync_copy(dst_hbm.at[tid], dst_ref)
    pltpu.sync_copy(ew_hbm.at[tid], ew_ref)

    def zewrows(r, carry):
        ewrows_ref[r, :] = zeros16f
        return carry
    lax.fori_loop(0, EBLK, zewrows, 0)
    pltpu.sync_copy(ewrows_ref.at[pl.ds(0, 40)],
                    deg_sh.at[pl.ds(tid * 40, 40)])
    plsc.subcore_barrier()

    def deg_blk(j, carry):
        for g in range(8):
            sl = pl.ds(g * 16, 16)
            d16 = dst_ref[j, sl]
            dstq_ref[0, sl] = d16 >> 4
            plsc.store_scatter(ewrows_ref, [iota + g * 16, d16 & 15],
                               ew_ref[j, sl])
        pltpu.sync_copy(ewrows_ref, deg_sh.at[dstq_ref.at[0]], add=True)
        for g in range(8):
            sl = pl.ds(g * 16, 16)
            d16 = dst_ref[j, sl]
            plsc.store_scatter(ewrows_ref, [iota + g * 16, d16 & 15],
                               zeros16f)
        return carry
    lax.fori_loop(cid * 79, 79 + cid * 80, deg_blk, 0)
    plsc.subcore_barrier()

    sl = pl.ds(tid * 40, 40)
    pltpu.sync_copy(deg_sh.at[sl], ewrows_ref.at[pl.ds(0, 40)])

    @pl.when(cid == 0)
    def _():
        pltpu.sync_copy(ewrows_ref.at[pl.ds(0, 40)], deg0_hbm.at[sl])

    @pl.when(cid == 1)
    def _():
        pltpu.sync_copy(ewrows_ref.at[pl.ds(0, 40)], deg1_hbm.at[sl])


def _kernel_b1(dstp1, ewp1):
    mesh = plsc.VectorSubcoreMesh(core_axis_name="c", subcore_axis_name="s")
    return pl.kernel(
        _kb1_body,
        out_type=[
            jax.ShapeDtypeStruct((DROWS, 16), jnp.float32),
            jax.ShapeDtypeStruct((DROWS, 16), jnp.float32),
        ],
        mesh=mesh,
        compiler_params=pltpu.CompilerParams(needs_layout_passes=False,
                                             use_tc_tiling_on_sc=False),
        scratch_types=[
            pltpu.VMEM((BLKS_PER_TILE, EBLK), jnp.int32),
            pltpu.VMEM((BLKS_PER_TILE, EBLK), jnp.float32),
            pltpu.VMEM((1, EBLK), jnp.int32),
            pltpu.VMEM((EBLK, 16), jnp.float32),
            pltpu.VMEM_SHARED((DROWS, 16), jnp.float32),
        ],
    )(dstp1, ewp1)


def _kd_body(d0_ref, d1_ref, dinv_ref):
    dinv_ref[...] = lax.rsqrt(d0_ref[...] + d1_ref[...] + 1.0)


def _kernel_d(deg0, deg1):
    return pl.pallas_call(
        _kd_body,
        out_shape=jax.ShapeDtypeStruct((DROWS, 16), jnp.float32),
    )(deg0, deg1)


def _kb2_body(edata_hbm, xlo_hbm, xhi_hbm, acc_hbm,
              edata_ref, rows_ref, acc_sh, sem_idx, sem_row, sem_sc):
    cid = lax.axis_index("c")
    tid = lax.axis_index("s")
    zeros16f = jnp.zeros((16,), jnp.float32)
    NG = EBLK2 // 16

    def zrows(r, carry):
        for jj in range(10):
            rows_ref[0, r, pl.ds(jj * 16, 16)] = zeros16f
        return carry
    lax.fori_loop(0, EBLK2, zrows, 0)
    base = tid * 625
    for i in range(13):
        pltpu.sync_copy(rows_ref.at[0], acc_sh.at[pl.ds(base + i * 48, 48)])
    pltpu.sync_copy(rows_ref.at[0].at[pl.ds(0, 1)],
                    acc_sh.at[pl.ds(base + 624, 1)])
    plsc.subcore_barrier()

    def fetch(jb):
        r = lax.rem(jb, 6)
        pltpu.async_copy(edata_hbm.at[tid, jb], edata_ref.at[r],
                         sem_idx.at[r])

    def fetch_wait(jb):
        r = lax.rem(jb, 6)
        pltpu.make_async_copy(edata_hbm.at[tid, jb], edata_ref.at[r],
                              sem_idx.at[r]).wait()

    def gather_issue(jb):
        r = lax.rem(jb, 6)
        b = lax.rem(jb, 3)

        @pl.when(cid == 0)
        def _():
            pltpu.async_copy(xlo_hbm.at[edata_ref.at[r, 0]], rows_ref.at[b],
                             sem_row.at[b])

        @pl.when(cid == 1)
        def _():
            pltpu.async_copy(xhi_hbm.at[edata_ref.at[r, 0]], rows_ref.at[b],
                             sem_row.at[b])

    def gather_wait(jb):
        r = lax.rem(jb, 6)
        b = lax.rem(jb, 3)
        pltpu.make_async_copy(xlo_hbm.at[edata_ref.at[r, 0]], rows_ref.at[b],
                              sem_row.at[b]).wait()

    def scatter_issue(jb):
        r = lax.rem(jb, 6)
        b = lax.rem(jb, 3)
        pltpu.async_copy(rows_ref.at[b], acc_sh.at[edata_ref.at[r, 1]],
                         sem_sc.at[b], add=True)

    def scatter_wait(jb):
        r = lax.rem(jb, 6)
        b = lax.rem(jb, 3)
        pltpu.make_async_copy(rows_ref.at[b], acc_sh.at[edata_ref.at[r, 1]],
                              sem_sc.at[b]).wait()

    fetch(0)
    fetch(1)
    fetch(2)
    fetch_wait(0)
    gather_issue(0)
    fetch_wait(1)
    gather_issue(1)

    def edge_blk(j, carry):
        r = lax.rem(j, 6)
        b = lax.rem(j, 3)
        gather_wait(j)

        for g in range(NG):
            ewv = plsc.bitcast(edata_ref[r, 2, pl.ds(g * 16, 16)],
                               jnp.float32)
            for lane in range(16):
                s = ewv[lane]
                row = g * 16 + lane
                for jj in range(10):
                    sl2 = pl.ds(jj * 16, 16)
                    rows_ref[b, row, sl2] = rows_ref[b, row, sl2] * s
        scatter_issue(j)

        @pl.when(j + 2 < NBLK2)
        def _():
            fetch_wait(j + 2)

            @pl.when(j >= 1)
            def _():
                scatter_wait(j - 1)
            gather_issue(j + 2)

            @pl.when(j + 3 < NBLK2)
            def _():
                fetch(j + 3)
        return carry
    lax.fori_loop(0, NBLK2, edge_blk, 0)

    for jb in range(NBLK2 - 3, NBLK2):
        scatter_wait(jb)
    plsc.subcore_barrier()

    hbase = cid * ACCR + tid * 625
    for i in range(13):
        pltpu.sync_copy(acc_sh.at[pl.ds(base + i * 48, 48)], rows_ref.at[0])
        pltpu.sync_copy(rows_ref.at[0],
                        acc_hbm.at[pl.ds(hbase + i * 48, 48)])
    pltpu.sync_copy(acc_sh.at[pl.ds(base + 624, 1)],
                    rows_ref.at[0].at[pl.ds(0, 1)])
    pltpu.sync_copy(rows_ref.at[0].at[pl.ds(0, 1)],
                    acc_hbm.at[pl.ds(hbase + 624, 1)])


def _kernel_b2(edata, xl_lo, xl_hi):
    mesh = plsc.VectorSubcoreMesh(core_axis_name="c", subcore_axis_name="s")
    return pl.kernel(
        _kb2_body,
        out_type=[
            jax.ShapeDtypeStruct((2 * ACCR, HALF), jnp.float32),
        ],
        mesh=mesh,
        compiler_params=pltpu.CompilerParams(needs_layout_passes=False,
                                             use_tc_tiling_on_sc=False),
        scratch_types=[
            pltpu.VMEM((6, 3, EBLK2), jnp.int32),
            pltpu.VMEM((3, EBLK2, HALF), jnp.float32),
            pltpu.VMEM_SHARED((ACCR, HALF), jnp.float32),
            pltpu.SemaphoreType.DMA((6,)),
            pltpu.SemaphoreType.DMA((3,)),
            pltpu.SemaphoreType.DMA((3,)),
        ],
    )(edata, xl_lo, xl_hi)


def _kc_body(alo_ref, ahi_ref, xlo_ref, xhi_ref, dinv_ref, gb_ref, w2_ref,
             b2_ref, out_ref):
    dv = dinv_ref[...]
    gb = gb_ref[...]
    glo = jnp.maximum(dv * (alo_ref[...] + xlo_ref[...]) + gb, 0.0)
    ghi = jnp.maximum(dv * (ahi_ref[...] + xhi_ref[...]) + gb, 0.0)
    z = (lax.dot_general(glo, w2_ref[:HALF], (((1,), (0,)), ((), ())),
                         preferred_element_type=jnp.float32)
         + lax.dot_general(ghi, w2_ref[HALF:], (((1,), (0,)), ((), ())),
                           preferred_element_type=jnp.float32))
    b2 = b2_ref[...]
    for t in range(TP2):
        zt = z[:, t * 256:(t + 1) * 256] + b2
        out_ref[t] = zt[:, :HID] * jax.nn.sigmoid(zt[:, HID:])


def _kernel_c(acc_cat, xl_lo, xl_hi, dinv2, gb160, w2big, b2):
    return pl.pallas_call(
        _kc_body,
        grid=(N // BN,),
        in_specs=[
            pl.BlockSpec((BN, HALF), lambda i: (i, 0)),
            pl.BlockSpec((BN, HALF), lambda i: (i + ACCR // BN, 0)),
            pl.BlockSpec((BN, HALF), lambda i: (i, 0)),
            pl.BlockSpec((BN, HALF), lambda i: (i, 0)),
            pl.BlockSpec((BN, 1), lambda i: (i, 0)),
            pl.BlockSpec((1, HALF), lambda i: (0, 0)),
            pl.BlockSpec((2 * HALF, TP2 * 256), lambda i: (0, 0)),
            pl.BlockSpec((1, 256), lambda i: (0, 0)),
        ],
        out_specs=pl.BlockSpec((TP2, BN, HID), lambda i: (0, i, 0)),
        out_shape=jax.ShapeDtypeStruct((TP2, N, HID), jnp.float32),
    )(acc_cat, acc_cat, xl_lo, xl_hi, dinv2, gb160, w2big, b2)


@jax.jit
def kernel(x, edge_index, edge_attr, batch, conv1_w, conv1_b, gcn_w, gcn_b,
           conv2_w, conv2_b):
    del batch
    xt = jnp.transpose(x, (0, 2, 1))
    w1 = jnp.transpose(conv1_w, (2, 1, 0))
    b1 = conv1_b[None, :]

    pad = EPAD - E
    fill = jnp.arange(pad, dtype=jnp.int32) % N
    src_f = jnp.concatenate([edge_index[0], fill])
    dst_f = jnp.concatenate([edge_index[1], fill])
    ew_f = jnp.concatenate([edge_attr, jnp.zeros((pad,), jnp.float32)])
    srcp = src_f.reshape(NTILES, NBLK2, EBLK2)
    dstp = dst_f.reshape(NTILES, NBLK2, EBLK2)
    ewp = lax.bitcast_convert_type(ew_f, jnp.int32).reshape(
        NTILES, NBLK2, EBLK2)
    edata = jnp.stack([srcp, dstp, ewp], axis=2)
    dstp1 = dst_f.reshape(NTILES, BLKS_PER_TILE, EBLK)
    ewp1 = ew_f.reshape(NTILES, BLKS_PER_TILE, EBLK)

    deg0, deg1 = _kernel_b1(dstp1, ewp1)
    dinv = _kernel_d(deg0, deg1).reshape(NPAD)
    dinv2 = dinv[:N, None]

    xl_lo, xl_hi = _kernel_a(xt, w1, b1, gcn_w, dinv2)
    (acc_cat,) = _kernel_b2(edata, xl_lo, xl_hi)

    gb160 = jnp.tile(gcn_b, 5)[None, :]

    w2t = jnp.transpose(conv2_w, (2, 1, 0))
    w2big = jnp.zeros((2 * HALF, TP2 * 256), jnp.float32)
    for t in range(TP2):
        for k in range(3):
            w2big = w2big.at[(t + k) * GCN_OUT:(t + k + 1) * GCN_OUT,
                             t * 256:(t + 1) * 256].set(w2t[k])

    out8 = _kernel_c(acc_cat, xl_lo, xl_hi, dinv2, gb160, w2big,
                     conv2_b[None, :])
    return jnp.transpose(out8, (1, 2, 0))
